# baseline (speedup 1.0000x reference)
import functools

import jax
import jax.numpy as jnp
from jax import lax
from jax.experimental import pallas as pl
from jax.experimental.pallas import tpu as pltpu
from jax.experimental.pallas import tpu_sc as plsc

N = 100000
E = 1600000
NUM_OPS = 1000
NUM_GRAPHS = 256
EMB = 16
XNUM = 5
HID = 64
NCLS = 2

BN = 1024
NCHUNK = 98
NPAD = BN * NCHUNK
ROWS_PER_TILE_SLICE = NPAD // 16
EROW = 128
EPAD = 16 * 98 * 8 * EROW
NEROW = EPAD // EROW
F32 = jnp.float32

_mesh = plsc.VectorSubcoreMesh(core_axis_name="c", subcore_axis_name="s")


@functools.partial(
    pl.kernel,
    out_type=jax.ShapeDtypeStruct((2, NPAD), F32),
    mesh=_mesh,
    scratch_types=[
        pltpu.VMEM((1024,), jnp.int32),
        pltpu.VMEM((1024,), F32),
        pltpu.VMEM((ROWS_PER_TILE_SLICE,), F32),
        pltpu.VMEM_SHARED((NPAD,), F32),
    ],
)
def _deg_kernel(dst_hbm, out_hbm, dstv, ones_v, zbuf, acc):
    c = lax.axis_index("c")
    s = lax.axis_index("s")

    def of(i, _):
        ones_v[pl.ds(i * 16, 16)] = jnp.ones((16,), F32)
        return 0

    lax.fori_loop(0, 1024 // 16, of, 0)

    def zf(i, _):
        zbuf[pl.ds(i * 16, 16)] = jnp.zeros((16,), F32)
        return 0

    lax.fori_loop(0, ROWS_PER_TILE_SLICE // 16, zf, 0)
    pltpu.sync_copy(zbuf, acc.at[pl.ds(s * ROWS_PER_TILE_SLICE, ROWS_PER_TILE_SLICE)])
    plsc.subcore_barrier()

    base = c * (EPAD // 2) + s * (EPAD // 32)

    def body(i, _):
        pltpu.sync_copy(dst_hbm.at[pl.ds(base + i * 1024, 1024)], dstv)
        pltpu.sync_copy(ones_v, acc.at[dstv], add=True)
        return 0

    lax.fori_loop(0, EPAD // 32 // 1024, body, 0)
    plsc.subcore_barrier()
    pltpu.sync_copy(
        acc.at[pl.ds(s * ROWS_PER_TILE_SLICE, ROWS_PER_TILE_SLICE)],
        out_hbm.at[c, pl.ds(s * ROWS_PER_TILE_SLICE, ROWS_PER_TILE_SLICE)],
    )


EC = 512
NCH_TILE = EPAD // 16 // EC


@functools.partial(
    pl.kernel,
    out_type=jax.ShapeDtypeStruct((4, NPAD, 16), F32),
    mesh=_mesh,
    compiler_params=pltpu.CompilerParams(use_tc_tiling_on_sc=False),
    scratch_types=[
        pltpu.VMEM((2, EC), jnp.int32),
        pltpu.VMEM((2, EC), jnp.int32),
        pltpu.VMEM((2, EC, 16), F32),
        pltpu.VMEM_SHARED((NPAD, 16), F32),
        pltpu.SemaphoreType.DMA,
        pltpu.SemaphoreType.DMA,
    ],
)
def _msg_kernel(y_hbm, src_hbm, dst_hbm, out_hbm, srcv, dstv, rows, acc,
                gsem, ssem):
    c = lax.axis_index("c")
    s = lax.axis_index("s")
    lo = s * ROWS_PER_TILE_SLICE
    ebase = s * (EPAD // 16)

    def load_idx(it, b):
        pltpu.sync_copy(src_hbm.at[pl.ds(ebase + it * EC, EC)], srcv.at[b])
        pltpu.sync_copy(dst_hbm.at[pl.ds(ebase + it * EC, EC)], dstv.at[b])

    def cg_pass(k, _):
        cg = 2 * k + c
        pltpu.sync_copy(
            y_hbm.at[cg, pl.ds(lo, ROWS_PER_TILE_SLICE)],
            acc.at[pl.ds(lo, ROWS_PER_TILE_SLICE)],
        )
        plsc.subcore_barrier()

        load_idx(0, 0)
        pltpu.async_copy(y_hbm.at[cg].at[srcv.at[0]], rows.at[0], gsem)

        def body(i, _):
            for b in range(2):
                it = 2 * i + b
                nb = 1 - b
                pltpu.make_async_copy(
                    y_hbm.at[cg].at[srcv.at[b]], rows.at[b], gsem
                ).wait()
                if b == 1:
                    pltpu.make_async_copy(
                        rows.at[nb], acc.at[dstv.at[nb]], ssem
                    ).wait()
                else:
                    @pl.when(i >= 1)
                    def _():
                        pltpu.make_async_copy(
                            rows.at[nb], acc.at[dstv.at[nb]], ssem
                        ).wait()
                if b == 0:
                    load_idx(it + 1, nb)
                    pltpu.async_copy(
                        y_hbm.at[cg].at[srcv.at[nb]], rows.at[nb], gsem
                    )
                else:
                    @pl.when(i <= (NCH_TILE // 2) - 2)
                    def _():
                        load_idx(it + 1, nb)
                        pltpu.async_copy(
                            y_hbm.at[cg].at[srcv.at[nb]], rows.at[nb], gsem
                        )
                pltpu.async_copy(rows.at[b], acc.at[dstv.at[b]], ssem, add=True)
            return 0

        lax.fori_loop(0, NCH_TILE // 2, body, 0)
        pltpu.make_async_copy(rows.at[1], acc.at[dstv.at[1]], ssem).wait()
        plsc.subcore_barrier()
        pltpu.sync_copy(
            acc.at[pl.ds(lo, ROWS_PER_TILE_SLICE)],
            out_hbm.at[cg, pl.ds(lo, ROWS_PER_TILE_SLICE)],
        )
        plsc.subcore_barrier()
        return 0

    lax.fori_loop(0, 2, cg_pass, 0)


def _prep_body(deg_ref, x_ref, opi_ref, emb_ref, w1a_ref, w1b_ref, y_ref):
    p = deg_ref[0] + deg_ref[1] + 1.0
    dinv = lax.rsqrt(p)
    xw = jnp.dot(x_ref[...], w1a_ref[0], preferred_element_type=F32)
    embw = jnp.dot(emb_ref[...], w1b_ref[0], preferred_element_type=F32)
    oh = (opi_ref[...][:, None]
          == lax.broadcasted_iota(jnp.int32, (BN, NUM_OPS), 1)).astype(F32)
    xw = xw + jnp.dot(oh, embw, preferred_element_type=F32)
    y_ref[0] = xw * dinv[:, None]


def _prep_call(deg2, xp, opp, op_emb, w1a, w1b):
    return pl.pallas_call(
        _prep_body,
        grid=(4, NCHUNK),
        in_specs=[
            pl.BlockSpec((2, BN), lambda cg, ch: (0, ch)),
            pl.BlockSpec((BN, XNUM), lambda cg, ch: (ch, 0)),
            pl.BlockSpec((BN,), lambda cg, ch: (ch,)),
            pl.BlockSpec((NUM_OPS, EMB), lambda cg, ch: (0, 0)),
            pl.BlockSpec((1, XNUM, 16), lambda cg, ch: (cg, 0, 0)),
            pl.BlockSpec((1, EMB, 16), lambda cg, ch: (cg, 0, 0)),
        ],
        out_specs=pl.BlockSpec((1, BN, 16), lambda cg, ch: (cg, ch, 0)),
        out_shape=jax.ShapeDtypeStruct((4, NPAD, 16), F32),
    )(deg2, xp, opp, op_emb, w1a, w1b)


def _mid_body(deg_ref, z_ref, b1_ref, w2_ref, y_ref):
    p = deg_ref[0] + deg_ref[1] + 1.0
    dinv = lax.rsqrt(p)
    accum = jnp.zeros((BN, 16), F32)
    for ci in range(4):
        h = jnp.maximum(z_ref[ci] * dinv[:, None] + b1_ref[ci][None, :], 0.0)
        accum = accum + jnp.dot(
            h, w2_ref[0, ci * 16:(ci + 1) * 16, :], preferred_element_type=F32
        )
    y_ref[0] = accum * dinv[:, None]


def _mid_call(deg2, z1, b1r, w2r):
    return pl.pallas_call(
        _mid_body,
        grid=(4, NCHUNK),
        in_specs=[
            pl.BlockSpec((2, BN), lambda cg, ch: (0, ch)),
            pl.BlockSpec((4, BN, 16), lambda cg, ch: (0, ch, 0)),
            pl.BlockSpec((4, 16), lambda cg, ch: (0, 0)),
            pl.BlockSpec((1, HID, 16), lambda cg, ch: (cg, 0, 0)),
        ],
        out_specs=pl.BlockSpec((1, BN, 16), lambda cg, ch: (cg, ch, 0)),
        out_shape=jax.ShapeDtypeStruct((4, NPAD, 16), F32),
    )(deg2, z1, b1r, w2r)


def _pool_body(deg_ref, z_ref, b2_ref, batch_ref, wlin_ref, blin_ref,
               out_ref, pooled, cnt):
    ch = pl.program_id(0)

    @pl.when(ch == 0)
    def _():
        pooled[...] = jnp.zeros_like(pooled)
        cnt[...] = jnp.zeros_like(cnt)

    p = deg_ref[0] + deg_ref[1] + 1.0
    dinv = lax.rsqrt(p)
    oht = (lax.broadcasted_iota(jnp.int32, (NUM_GRAPHS, BN), 0)
           == batch_ref[...][None, :]).astype(F32)
    cnt[0, :] += jnp.sum(oht, axis=1)
    for cg in range(4):
        h = jnp.maximum(z_ref[cg] * dinv[:, None] + b2_ref[cg][None, :], 0.0)
        pooled[:, cg * 16:(cg + 1) * 16] += jnp.dot(
            oht, h, preferred_element_type=F32
        )

    @pl.when(ch == NCHUNK - 1)
    def _():
        c = jnp.maximum(cnt[0, :], 1.0)
        pm = pooled[...] / c[:, None]
        out_ref[...] = (
            jnp.dot(pm, wlin_ref[...], preferred_element_type=F32)
            + blin_ref[0][None, :]
        )


def _pool_call(deg2, z2, b2r, batchp, wlin, blinr):
    return pl.pallas_call(
        _pool_body,
        grid=(NCHUNK,),
        in_specs=[
            pl.BlockSpec((2, BN), lambda ch: (0, ch)),
            pl.BlockSpec((4, BN, 16), lambda ch: (0, ch, 0)),
            pl.BlockSpec((4, 16), lambda ch: (0, 0)),
            pl.BlockSpec((BN,), lambda ch: (ch,)),
            pl.BlockSpec((HID, NCLS), lambda ch: (0, 0)),
            pl.BlockSpec((1, NCLS), lambda ch: (0, 0)),
        ],
        out_specs=pl.BlockSpec((NUM_GRAPHS, NCLS), lambda ch: (0, 0)),
        out_shape=jax.ShapeDtypeStruct((NUM_GRAPHS, NCLS), F32),
        scratch_shapes=[
            pltpu.VMEM((NUM_GRAPHS, HID), F32),
            pltpu.VMEM((1, NUM_GRAPHS), F32),
        ],
    )(deg2, z2, b2r, batchp, wlin, blinr)


def kernel(x, op_idx, edge_index, batch, op_emb, W1, b1, W2, b2, Wlin, blin):
    npx = NPAD - N
    xp = jnp.concatenate([x, jnp.zeros((npx, XNUM), F32)], axis=0)
    opp = jnp.concatenate(
        [op_idx.astype(jnp.int32), jnp.zeros((npx,), jnp.int32)]
    )
    batchp = jnp.concatenate(
        [batch.astype(jnp.int32), jnp.full((npx,), NUM_GRAPHS + 1, jnp.int32)]
    )
    pad_ids = N + (jnp.arange(EPAD - E, dtype=jnp.int32) % npx)
    srcp = jnp.concatenate([edge_index[0].astype(jnp.int32), pad_ids])
    dstp = jnp.concatenate([edge_index[1].astype(jnp.int32), pad_ids])

    w1a = W1[:XNUM].reshape(XNUM, 4, 16).transpose(1, 0, 2)
    w1b = W1[XNUM:].reshape(EMB, 4, 16).transpose(1, 0, 2)
    w2r = W2.reshape(HID, 4, 16).transpose(1, 0, 2)

    deg2 = _deg_kernel(dstp)
    y1 = _prep_call(deg2, xp, opp, op_emb, w1a, w1b)
    z1 = _msg_kernel(y1, srcp, dstp)
    y2 = _mid_call(deg2, z1, b1.reshape(4, 16), w2r)
    z2 = _msg_kernel(y2, srcp, dstp)
    return _pool_call(deg2, z2, b2.reshape(4, 16), batchp, Wlin,
                      blin.reshape(1, NCLS))


# speedup vs baseline: 1.2905x; 1.2905x over previous
import functools

import jax
import jax.numpy as jnp
from jax import lax
from jax.experimental import pallas as pl
from jax.experimental.pallas import tpu as pltpu
from jax.experimental.pallas import tpu_sc as plsc

N = 100000
E = 1600000
NUM_OPS = 1000
NUM_GRAPHS = 256
EMB = 16
XNUM = 5
HID = 64
NCLS = 2

BN = 1024
NCHUNK = 98
NPAD = BN * NCHUNK
ROWS_PER_TILE_SLICE = NPAD // 16
EROW = 128
EPAD = 16 * 98 * 8 * EROW
NEROW = EPAD // EROW
F32 = jnp.float32

_mesh = plsc.VectorSubcoreMesh(core_axis_name="c", subcore_axis_name="s")


@functools.partial(
    pl.kernel,
    out_type=jax.ShapeDtypeStruct((2, NPAD), F32),
    mesh=_mesh,
    scratch_types=[
        pltpu.VMEM((8, EROW), jnp.int32),
        pltpu.VMEM((EROW,), F32),
        pltpu.VMEM((ROWS_PER_TILE_SLICE,), F32),
        pltpu.VMEM_SHARED((NPAD,), F32),
    ],
)
def _deg_kernel(dst_hbm, out_hbm, dstv, ones_v, zbuf, acc):
    c = lax.axis_index("c")
    s = lax.axis_index("s")
    for i in range(EROW // 16):
        ones_v[pl.ds(i * 16, 16)] = jnp.ones((16,), F32)

    def zf(i, _):
        zbuf[pl.ds(i * 16, 16)] = jnp.zeros((16,), F32)
        return 0

    lax.fori_loop(0, ROWS_PER_TILE_SLICE // 16, zf, 0)
    pltpu.sync_copy(zbuf, acc.at[pl.ds(s * ROWS_PER_TILE_SLICE, ROWS_PER_TILE_SLICE)])
    plsc.subcore_barrier()

    base_row = c * (NEROW // 2) + s * (NEROW // 32)

    def body(i, _):
        pltpu.sync_copy(dst_hbm.at[pl.ds(base_row + i * 8, 8)], dstv)
        for j in range(8):
            pltpu.sync_copy(ones_v, acc.at[dstv.at[j]], add=True)
        return 0

    lax.fori_loop(0, NEROW // 32 // 8, body, 0)
    plsc.subcore_barrier()
    pltpu.sync_copy(
        acc.at[pl.ds(s * ROWS_PER_TILE_SLICE, ROWS_PER_TILE_SLICE)],
        out_hbm.at[c, pl.ds(s * ROWS_PER_TILE_SLICE, ROWS_PER_TILE_SLICE)],
    )


@functools.partial(
    pl.kernel,
    out_type=jax.ShapeDtypeStruct((4, NPAD, 16), F32),
    mesh=_mesh,
    compiler_params=pltpu.CompilerParams(use_tc_tiling_on_sc=False),
    scratch_types=[
        pltpu.VMEM((8, EROW), jnp.int32),
        pltpu.VMEM((8, EROW), jnp.int32),
        pltpu.VMEM((8 * EROW, 16), F32),
        pltpu.VMEM_SHARED((NPAD, 16), F32),
        pltpu.SemaphoreType.DMA,
    ],
)
def _msg_kernel(y_hbm, src_hbm, dst_hbm, out_hbm, srcv, dstv, rows, acc, sem):
    c = lax.axis_index("c")
    s = lax.axis_index("s")
    lo = s * ROWS_PER_TILE_SLICE

    def cg_pass(k, _):
        cg = 2 * k + c
        pltpu.sync_copy(
            y_hbm.at[cg, pl.ds(lo, ROWS_PER_TILE_SLICE)],
            acc.at[pl.ds(lo, ROWS_PER_TILE_SLICE)],
        )
        plsc.subcore_barrier()

        def body(i, _):
            r = s * (NEROW // 16) + i * 8
            pltpu.sync_copy(src_hbm.at[pl.ds(r, 8)], srcv)
            pltpu.sync_copy(dst_hbm.at[pl.ds(r, 8)], dstv)
            copies = []
            for j in range(8):
                copies.append(
                    pltpu.async_copy(
                        y_hbm.at[cg].at[srcv.at[j]],
                        rows.at[pl.ds(j * EROW, EROW)],
                        sem,
                    )
                )
            for cp in copies:
                cp.wait()
            for j in range(8):
                pltpu.sync_copy(
                    rows.at[pl.ds(j * EROW, EROW)], acc.at[dstv.at[j]], add=True
                )
            return 0

        lax.fori_loop(0, NEROW // 16 // 8, body, 0)
        plsc.subcore_barrier()
        pltpu.sync_copy(
            acc.at[pl.ds(lo, ROWS_PER_TILE_SLICE)],
            out_hbm.at[cg, pl.ds(lo, ROWS_PER_TILE_SLICE)],
        )
        plsc.subcore_barrier()
        return 0

    lax.fori_loop(0, 2, cg_pass, 0)


def _prep_body(deg_ref, x_ref, opi_ref, emb_ref, w1a_ref, w1b_ref, y_ref):
    p = deg_ref[0] + deg_ref[1] + 1.0
    dinv = lax.rsqrt(p)
    xw = jnp.dot(x_ref[...], w1a_ref[...], preferred_element_type=F32)
    embw = jnp.dot(emb_ref[...], w1b_ref[...], preferred_element_type=F32)
    oh = (opi_ref[...][:, None]
          == lax.broadcasted_iota(jnp.int32, (BN, NUM_OPS), 1)).astype(F32)
    xw = xw + jnp.dot(oh, embw, preferred_element_type=F32)
    xw = xw * dinv[:, None]
    for cg in range(4):
        y_ref[cg] = xw[:, cg * 16:(cg + 1) * 16]


def _prep_call(deg2, xp, opp, op_emb, w1a, w1b):
    return pl.pallas_call(
        _prep_body,
        grid=(NCHUNK,),
        in_specs=[
            pl.BlockSpec((2, BN), lambda ch: (0, ch)),
            pl.BlockSpec((BN, XNUM), lambda ch: (ch, 0)),
            pl.BlockSpec((BN,), lambda ch: (ch,)),
            pl.BlockSpec((NUM_OPS, EMB), lambda ch: (0, 0)),
            pl.BlockSpec((XNUM, HID), lambda ch: (0, 0)),
            pl.BlockSpec((EMB, HID), lambda ch: (0, 0)),
        ],
        out_specs=pl.BlockSpec((4, BN, 16), lambda ch: (0, ch, 0)),
        out_shape=jax.ShapeDtypeStruct((4, NPAD, 16), F32),
    )(deg2, xp, opp, op_emb, w1a, w1b)


def _mid_body(deg_ref, z_ref, b1_ref, w2_ref, y_ref):
    p = deg_ref[0] + deg_ref[1] + 1.0
    dinv = lax.rsqrt(p)
    h = jnp.concatenate([z_ref[ci] for ci in range(4)], axis=1)
    h = jnp.maximum(h * dinv[:, None] + b1_ref[0][None, :], 0.0)
    o = jnp.dot(h, w2_ref[...], preferred_element_type=F32) * dinv[:, None]
    for cg in range(4):
        y_ref[cg] = o[:, cg * 16:(cg + 1) * 16]


def _mid_call(deg2, z1, b1r, w2r):
    return pl.pallas_call(
        _mid_body,
        grid=(NCHUNK,),
        in_specs=[
            pl.BlockSpec((2, BN), lambda ch: (0, ch)),
            pl.BlockSpec((4, BN, 16), lambda ch: (0, ch, 0)),
            pl.BlockSpec((1, HID), lambda ch: (0, 0)),
            pl.BlockSpec((HID, HID), lambda ch: (0, 0)),
        ],
        out_specs=pl.BlockSpec((4, BN, 16), lambda ch: (0, ch, 0)),
        out_shape=jax.ShapeDtypeStruct((4, NPAD, 16), F32),
    )(deg2, z1, b1r, w2r)


def _pool_body(deg_ref, z_ref, b2_ref, batch_ref, wlin_ref, blin_ref,
               out_ref, pooled, cnt):
    ch = pl.program_id(0)

    @pl.when(ch == 0)
    def _():
        pooled[...] = jnp.zeros_like(pooled)
        cnt[...] = jnp.zeros_like(cnt)

    p = deg_ref[0] + deg_ref[1] + 1.0
    dinv = lax.rsqrt(p)
    oht = (lax.broadcasted_iota(jnp.int32, (NUM_GRAPHS, BN), 0)
           == batch_ref[...][None, :]).astype(F32)
    cnt[0, :] += jnp.sum(oht, axis=1)
    h = jnp.concatenate([z_ref[ci] for ci in range(4)], axis=1)
    h = jnp.maximum(h * dinv[:, None] + b2_ref[0][None, :], 0.0)
    pooled[...] += jnp.dot(oht, h, preferred_element_type=F32)

    @pl.when(ch == NCHUNK - 1)
    def _():
        c = jnp.maximum(cnt[0, :], 1.0)
        pm = pooled[...] / c[:, None]
        out_ref[...] = (
            jnp.dot(pm, wlin_ref[...], preferred_element_type=F32)
            + blin_ref[0][None, :]
        )


def _pool_call(deg2, z2, b2r, batchp, wlin, blinr):
    return pl.pallas_call(
        _pool_body,
        grid=(NCHUNK,),
        in_specs=[
            pl.BlockSpec((2, BN), lambda ch: (0, ch)),
            pl.BlockSpec((4, BN, 16), lambda ch: (0, ch, 0)),
            pl.BlockSpec((1, HID), lambda ch: (0, 0)),
            pl.BlockSpec((BN,), lambda ch: (ch,)),
            pl.BlockSpec((HID, NCLS), lambda ch: (0, 0)),
            pl.BlockSpec((1, NCLS), lambda ch: (0, 0)),
        ],
        out_specs=pl.BlockSpec((NUM_GRAPHS, NCLS), lambda ch: (0, 0)),
        out_shape=jax.ShapeDtypeStruct((NUM_GRAPHS, NCLS), F32),
        scratch_shapes=[
            pltpu.VMEM((NUM_GRAPHS, HID), F32),
            pltpu.VMEM((1, NUM_GRAPHS), F32),
        ],
    )(deg2, z2, b2r, batchp, wlin, blinr)


def kernel(x, op_idx, edge_index, batch, op_emb, W1, b1, W2, b2, Wlin, blin):
    npx = NPAD - N
    xp = jnp.concatenate([x, jnp.zeros((npx, XNUM), F32)], axis=0)
    opp = jnp.concatenate(
        [op_idx.astype(jnp.int32), jnp.zeros((npx,), jnp.int32)]
    )
    batchp = jnp.concatenate(
        [batch.astype(jnp.int32), jnp.full((npx,), NUM_GRAPHS + 1, jnp.int32)]
    )
    pad_ids = N + (jnp.arange(EPAD - E, dtype=jnp.int32) % npx)
    srcp = jnp.concatenate(
        [edge_index[0].astype(jnp.int32), pad_ids]
    ).reshape(NEROW, EROW)
    dstp = jnp.concatenate(
        [edge_index[1].astype(jnp.int32), pad_ids]
    ).reshape(NEROW, EROW)

    deg2 = _deg_kernel(dstp)
    y1 = _prep_call(deg2, xp, opp, op_emb, W1[:XNUM], W1[XNUM:])
    z1 = _msg_kernel(y1, srcp, dstp)
    y2 = _mid_call(deg2, z1, b1.reshape(1, HID), W2)
    z2 = _msg_kernel(y2, srcp, dstp)
    return _pool_call(deg2, z2, b2.reshape(1, HID), batchp, Wlin,
                      blin.reshape(1, NCLS))
